# baseline (speedup 1.0000x reference)
import jax
import jax.numpy as jnp
from jax import lax
from jax.experimental import pallas as pl
from jax.experimental.pallas import tpu as pltpu
from jax.experimental.pallas import tpu_sc as plsc

NC = 2
NS = 16
NW = NC * NS

D = 128
SEQ = 50
BATCH = 4096
B_TOT = BATCH * SEQ
BPW = B_TOT // NW
C = 200
NBUF = 4
NCH = BPW // C


def _emb_body(idx_hbm, w_hbm, out_hbm, idx_v, rows0, rows1, rows2, rows3,
              gsem0, gsem1, gsem2, gsem3, ssem0, ssem1, ssem2, ssem3):
    wid = lax.axis_index("s") * NC + lax.axis_index("c")
    base = wid * BPW
    pltpu.sync_copy(idx_hbm.at[pl.ds(base, BPW)], idx_v)

    rows = (rows0, rows1, rows2, rows3)
    gsem = (gsem0, gsem1, gsem2, gsem3)
    ssem = (ssem0, ssem1, ssem2, ssem3)
    gdesc = [None] * NBUF
    sdesc = [None] * NBUF
    for i in range(NCH):
        b = i % NBUF
        if i >= NBUF:
            sdesc[b].wait()
        gdesc[b] = pltpu.async_copy(
            w_hbm.at[idx_v.at[pl.ds(i * C, C)]], rows[b], gsem[b])
        if i >= 1:
            pb = (i - 1) % NBUF
            gdesc[pb].wait()
            sdesc[pb] = pltpu.async_copy(
                rows[pb], out_hbm.at[pl.ds(base + (i - 1) * C, C)], ssem[pb])
    last = (NCH - 1) % NBUF
    gdesc[last].wait()
    sdesc[last] = pltpu.async_copy(
        rows[last], out_hbm.at[pl.ds(base + (NCH - 1) * C, C)], ssem[last])
    for i in range(NCH - NBUF + 1, NCH):
        sdesc[i % NBUF].wait()


def kernel(indices, weight):
    idx_t = indices.T.reshape(-1).astype(jnp.int32)
    mesh = plsc.VectorSubcoreMesh(
        core_axis_name="c", subcore_axis_name="s",
        num_cores=NC, num_subcores=NS)
    out = pl.kernel(
        _emb_body,
        out_type=jax.ShapeDtypeStruct((B_TOT, D), jnp.float32),
        mesh=mesh,
        scratch_types=(
            [pltpu.VMEM((BPW,), jnp.int32)]
            + [pltpu.VMEM((C, D), jnp.float32)] * NBUF
            + [pltpu.SemaphoreType.DMA] * (2 * NBUF)
        ),
    )(idx_t, weight)
    return jnp.swapaxes(out.reshape(SEQ, BATCH, D), 0, 1)
